# baseline (speedup 1.0000x reference)
import functools

import jax
import jax.numpy as jnp
from jax import lax
from jax.experimental import pallas as pl
from jax.experimental.pallas import tpu as pltpu
from jax.experimental.pallas import tpu_sc as plsc

N = 10000
D = 128
H = 128
OUT = 64
CH = 128
NW = 32
NACC = 10240
RPT = NACC // 16
NB = RPT // CH
CG = 64
GRP = 16
NG = 10
RB = 1024
GRID = 10

_MESH = plsc.VectorSubcoreMesh(core_axis_name="c", subcore_axis_name="s")


def _hop_agg(table, srcs, dsts, zrow):
    @functools.partial(
        pl.kernel,
        out_type=jax.ShapeDtypeStruct((2, NACC, H), jnp.float32),
        mesh=_MESH,
        scratch_types=[
            pltpu.VMEM((CH, H), jnp.float32),
            pltpu.VMEM((GRP, CG), jnp.int32),
            pltpu.VMEM((GRP, CG), jnp.int32),
            pltpu.SemaphoreType.DMA,
            pltpu.SemaphoreType.DMA,
            pltpu.VMEM_SHARED((NACC, H), jnp.float32),
        ],
    )
    def k(table_h, srcs_h, dsts_h, zrow_h, ofeat_h,
          rows_v, src_v, dst_v, sem_a, sem_b, acc_s):
        c = lax.axis_index("c")
        s = lax.axis_index("s")
        w = c * 16 + s
        r0 = s * RPT
        pltpu.sync_copy(zrow_h, rows_v)
        for t in range(NB):
            pltpu.sync_copy(rows_v, acc_s.at[pl.ds(r0 + t * CH, CH)])
        plsc.subcore_barrier()

        rows_a = rows_v.at[pl.ds(0, CG)]
        rows_b = rows_v.at[pl.ds(CG, CG)]

        @pl.loop(0, NG)
        def _(g):
            pltpu.sync_copy(srcs_h.at[w * NG + g], src_v)
            pltpu.sync_copy(dsts_h.at[w * NG + g], dst_v)
            pltpu.async_copy(table_h.at[src_v.at[0]], rows_a, sem_a)

            @pl.loop(0, GRP // 2)
            def _(p):
                pltpu.make_async_copy(zrow_h.at[pl.ds(0, CG)],
                                      rows_a, sem_a).wait()
                pltpu.async_copy(table_h.at[src_v.at[2 * p + 1]],
                                 rows_b, sem_b)
                pltpu.sync_copy(rows_a, acc_s.at[dst_v.at[2 * p]], add=True)
                pltpu.make_async_copy(zrow_h.at[pl.ds(0, CG)],
                                      rows_b, sem_b).wait()

                @pl.when(p < GRP // 2 - 1)
                def _():
                    pltpu.async_copy(table_h.at[src_v.at[2 * p + 2]],
                                     rows_a, sem_a)

                pltpu.sync_copy(rows_b, acc_s.at[dst_v.at[2 * p + 1]],
                                add=True)

        plsc.subcore_barrier()
        for t in range(NB):
            pltpu.sync_copy(acc_s.at[pl.ds(r0 + t * CH, CH)], rows_v)
            pltpu.sync_copy(rows_v, ofeat_h.at[c, pl.ds(r0 + t * CH, CH)])

    return k(table, srcs, dsts, zrow)


def _cnt_body(d_ref, o_ref):
    d = d_ref[...]
    q = jax.lax.shift_right_logical(d, 7)
    r = jax.lax.bitwise_and(d, 127)
    lanes = jax.lax.broadcasted_iota(jnp.int32, (1, H), 1)
    a = jnp.where(q == lanes, 1.0, 0.0)
    b = jnp.where(r == lanes, 1.0, 0.0)
    part = jax.lax.dot_general(a, b, (((0,), (0,)), ((), ())),
                               preferred_element_type=jnp.float32)

    @pl.when(pl.program_id(0) == 0)
    def _():
        o_ref[...] = jnp.zeros_like(o_ref)

    o_ref[...] += part


EC = 8192


def _cnt_tc(dst_col):
    return pl.pallas_call(
        _cnt_body,
        grid=(dst_col.shape[0] // EC,),
        in_specs=[pl.BlockSpec((EC, 1), lambda i: (i, 0))],
        out_specs=pl.BlockSpec((H, H), lambda i: (0, 0)),
        out_shape=jax.ShapeDtypeStruct((H, H), jnp.float32),
    )(dst_col)


def _cnt_col(c_ref):
    m = c_ref[...]
    i0 = jax.lax.broadcasted_iota(jnp.int32, (RB, 8), 0) // H
    s0 = jax.lax.broadcasted_iota(jnp.int32, (RB, 8), 1)
    p = jnp.where(i0 == s0, 1.0, 0.0)
    y = jnp.dot(p, m, preferred_element_type=jnp.float32)
    i1 = jax.lax.broadcasted_iota(jnp.int32, (RB, H), 0) % H
    t1 = jax.lax.broadcasted_iota(jnp.int32, (RB, H), 1)
    qm = jnp.where(i1 == t1, 1.0, 0.0)
    return jnp.maximum(jnp.sum(y * qm, axis=1, keepdims=True), 1.0)


def _proj_body(x_ref, w_ref, b_ref, o_ref):
    o_ref[...] = jnp.maximum(
        jnp.dot(x_ref[...], w_ref[...], preferred_element_type=jnp.float32)
        + b_ref[...], 0.0)


def _proj(x, wT, b):
    return pl.pallas_call(
        _proj_body,
        grid=(GRID,),
        in_specs=[
            pl.BlockSpec((RB, D), lambda i: (i, 0)),
            pl.BlockSpec((D, H), lambda i: (0, 0)),
            pl.BlockSpec((1, H), lambda i: (0, 0)),
        ],
        out_specs=pl.BlockSpec((RB, H), lambda i: (i, 0)),
        out_shape=jax.ShapeDtypeStruct((N, H), jnp.float32),
    )(x, wT, b)


def _merge0_body(f_ref, c_ref, hp_ref, ha_ref, w0l_ref, b0_ref, w0r_ref,
                 w1r_ref, b1_ref, h_ref, r1_ref):
    mean = (f_ref[0] + f_ref[1]) / _cnt_col(c_ref)
    t = (jnp.dot(mean, w0l_ref[...], preferred_element_type=jnp.float32)
         + b0_ref[...]
         + jnp.dot(hp_ref[...], w0r_ref[...], preferred_element_type=jnp.float32))
    h_ref[...] = jnp.maximum(t, 0.0)
    r1_ref[...] = (jnp.dot(ha_ref[...], w1r_ref[...],
                           preferred_element_type=jnp.float32) + b1_ref[...])


def _merge0(feat, cnt, h_paper, h_author, w0lT, b0, w0rT, w1rT, b1):
    return pl.pallas_call(
        _merge0_body,
        grid=(GRID,),
        in_specs=[
            pl.BlockSpec((2, RB, H), lambda i: (0, i, 0)),
            pl.BlockSpec((8, H), lambda i: (i, 0)),
            pl.BlockSpec((RB, H), lambda i: (i, 0)),
            pl.BlockSpec((RB, H), lambda i: (i, 0)),
            pl.BlockSpec((H, H), lambda i: (0, 0)),
            pl.BlockSpec((1, H), lambda i: (0, 0)),
            pl.BlockSpec((H, H), lambda i: (0, 0)),
            pl.BlockSpec((H, OUT), lambda i: (0, 0)),
            pl.BlockSpec((1, OUT), lambda i: (0, 0)),
        ],
        out_specs=[
            pl.BlockSpec((RB, H), lambda i: (i, 0)),
            pl.BlockSpec((RB, OUT), lambda i: (i, 0)),
        ],
        out_shape=[
            jax.ShapeDtypeStruct((N, H), jnp.float32),
            jax.ShapeDtypeStruct((N, OUT), jnp.float32),
        ],
    )(feat, cnt, h_paper, h_author, w0lT, b0, w0rT, w1rT, b1)


def _final_body(f_ref, c_ref, r1_ref, w1l_ref, o_ref):
    mean = (f_ref[0] + f_ref[1]) / _cnt_col(c_ref)
    o_ref[...] = (jnp.dot(mean, w1l_ref[...],
                          preferred_element_type=jnp.float32) + r1_ref[...])


def _final(feat, cnt, r1, w1lT):
    return pl.pallas_call(
        _final_body,
        grid=(GRID,),
        in_specs=[
            pl.BlockSpec((2, RB, H), lambda i: (0, i, 0)),
            pl.BlockSpec((8, H), lambda i: (i, 0)),
            pl.BlockSpec((RB, OUT), lambda i: (i, 0)),
            pl.BlockSpec((H, OUT), lambda i: (0, 0)),
        ],
        out_specs=pl.BlockSpec((RB, OUT), lambda i: (i, 0)),
        out_shape=jax.ShapeDtypeStruct((N, OUT), jnp.float32),
    )(feat, cnt, r1, w1lT)


def kernel(x_author, x_paper, edge_index_hop0, edge_index_hop1,
           W_proj_author, b_proj_author, W_proj_paper, b_proj_paper,
           W0_l, b0_l, W0_r, W1_l, b1_l, W1_r):
    E = edge_index_hop0.shape[1]
    EP = NW * NG * GRP * CG
    npad = EP - E
    pad_src = (jnp.arange(npad, dtype=jnp.int32) * 37) % N
    pad_dst = N + (jnp.arange(npad, dtype=jnp.int32) % (NACC - N))

    def prep(idx, pad_vals):
        flat = jnp.concatenate([idx.astype(jnp.int32), pad_vals])
        return flat.reshape(NW * NG, GRP, CG), flat.reshape(EP, 1)

    src0, _ = prep(edge_index_hop0[0], pad_src)
    dst0, dcol0 = prep(edge_index_hop0[1], pad_dst)
    src1, _ = prep(edge_index_hop1[0], pad_src)
    dst1, dcol1 = prep(edge_index_hop1[1], pad_dst)

    zrow = jnp.zeros((CH, H), jnp.float32)

    h_author = _proj(x_author, W_proj_author.T, b_proj_author.reshape(1, H))
    h_paper = _proj(x_paper, W_proj_paper.T, b_proj_paper.reshape(1, H))

    cnt0 = _cnt_tc(dcol0)
    cnt1 = _cnt_tc(dcol1)

    feat0 = _hop_agg(h_author, src0, dst0, zrow)

    h, r1 = _merge0(feat0, cnt0, h_paper, h_author,
                    W0_l.T, b0_l.reshape(1, H), W0_r.T,
                    W1_r.T, b1_l.reshape(1, OUT))

    feat1 = _hop_agg(h, src1, dst1, zrow)

    return _final(feat1, cnt1, r1, W1_l.T)


# speedup vs baseline: 1.0859x; 1.0859x over previous
import functools

import jax
import jax.numpy as jnp
from jax import lax
from jax.experimental import pallas as pl
from jax.experimental.pallas import tpu as pltpu
from jax.experimental.pallas import tpu_sc as plsc

N = 10000
D = 128
H = 128
OUT = 64
CH = 128
NW = 32
NACC = 10240
RPT = NACC // 16
NB = RPT // CH
GRP = 8
NG = 10
RB = 1024
GRID = 10

_MESH = plsc.VectorSubcoreMesh(core_axis_name="c", subcore_axis_name="s")


def _hop_agg(table, srcs, dsts, zrow):
    @functools.partial(
        pl.kernel,
        out_type=jax.ShapeDtypeStruct((2, NACC, H), jnp.float32),
        mesh=_MESH,
        scratch_types=[
            pltpu.VMEM((CH, H), jnp.float32),
            pltpu.VMEM((GRP, CH), jnp.int32),
            pltpu.VMEM((GRP, CH), jnp.int32),
            pltpu.VMEM_SHARED((NACC, H), jnp.float32),
        ],
    )
    def k(table_h, srcs_h, dsts_h, zrow_h, ofeat_h,
          rows_v, src_v, dst_v, acc_s):
        c = lax.axis_index("c")
        s = lax.axis_index("s")
        w = c * 16 + s
        r0 = s * RPT
        pltpu.sync_copy(zrow_h, rows_v)
        for t in range(NB):
            pltpu.sync_copy(rows_v, acc_s.at[pl.ds(r0 + t * CH, CH)])
        plsc.subcore_barrier()

        @pl.loop(0, NG)
        def _(g):
            pltpu.sync_copy(srcs_h.at[w * NG + g], src_v)
            pltpu.sync_copy(dsts_h.at[w * NG + g], dst_v)

            @pl.loop(0, GRP)
            def _(j):
                pltpu.sync_copy(table_h.at[src_v.at[j]], rows_v)
                pltpu.sync_copy(rows_v, acc_s.at[dst_v.at[j]], add=True)

        plsc.subcore_barrier()
        for t in range(NB):
            pltpu.sync_copy(acc_s.at[pl.ds(r0 + t * CH, CH)], rows_v)
            pltpu.sync_copy(rows_v, ofeat_h.at[c, pl.ds(r0 + t * CH, CH)])

    return k(table, srcs, dsts, zrow)


def _cnt_body(d_ref, o_ref):
    d = d_ref[...]
    q = jax.lax.shift_right_logical(d, 7)
    r = jax.lax.bitwise_and(d, 127)
    lanes = jax.lax.broadcasted_iota(jnp.int32, (1, H), 1)
    a = jnp.where(q == lanes, 1.0, 0.0)
    b = jnp.where(r == lanes, 1.0, 0.0)
    part = jax.lax.dot_general(a, b, (((0,), (0,)), ((), ())),
                               preferred_element_type=jnp.float32)

    @pl.when(pl.program_id(0) == 0)
    def _():
        o_ref[...] = jnp.zeros_like(o_ref)

    o_ref[...] += part


EC = 8192


def _cnt_tc(dst_col):
    return pl.pallas_call(
        _cnt_body,
        grid=(dst_col.shape[0] // EC,),
        in_specs=[pl.BlockSpec((EC, 1), lambda i: (i, 0))],
        out_specs=pl.BlockSpec((H, H), lambda i: (0, 0)),
        out_shape=jax.ShapeDtypeStruct((H, H), jnp.float32),
    )(dst_col)


def _cnt_col(c_ref):
    m = c_ref[...]
    i0 = jax.lax.broadcasted_iota(jnp.int32, (RB, 8), 0) // H
    s0 = jax.lax.broadcasted_iota(jnp.int32, (RB, 8), 1)
    p = jnp.where(i0 == s0, 1.0, 0.0)
    y = jnp.dot(p, m, preferred_element_type=jnp.float32)
    i1 = jax.lax.broadcasted_iota(jnp.int32, (RB, H), 0) % H
    t1 = jax.lax.broadcasted_iota(jnp.int32, (RB, H), 1)
    qm = jnp.where(i1 == t1, 1.0, 0.0)
    return jnp.maximum(jnp.sum(y * qm, axis=1, keepdims=True), 1.0)


def _proj_body(x_ref, w_ref, b_ref, o_ref):
    o_ref[...] = jnp.maximum(
        jnp.dot(x_ref[...], w_ref[...], preferred_element_type=jnp.float32)
        + b_ref[...], 0.0)


def _proj(x, wT, b):
    return pl.pallas_call(
        _proj_body,
        grid=(GRID,),
        in_specs=[
            pl.BlockSpec((RB, D), lambda i: (i, 0)),
            pl.BlockSpec((D, H), lambda i: (0, 0)),
            pl.BlockSpec((1, H), lambda i: (0, 0)),
        ],
        out_specs=pl.BlockSpec((RB, H), lambda i: (i, 0)),
        out_shape=jax.ShapeDtypeStruct((N, H), jnp.float32),
    )(x, wT, b)


def _merge0_body(f_ref, c_ref, hp_ref, ha_ref, w0l_ref, b0_ref, w0r_ref,
                 w1r_ref, b1_ref, h_ref, r1_ref):
    mean = (f_ref[0] + f_ref[1]) / _cnt_col(c_ref)
    t = (jnp.dot(mean, w0l_ref[...], preferred_element_type=jnp.float32)
         + b0_ref[...]
         + jnp.dot(hp_ref[...], w0r_ref[...], preferred_element_type=jnp.float32))
    h_ref[...] = jnp.maximum(t, 0.0)
    r1_ref[...] = (jnp.dot(ha_ref[...], w1r_ref[...],
                           preferred_element_type=jnp.float32) + b1_ref[...])


def _merge0(feat, cnt, h_paper, h_author, w0lT, b0, w0rT, w1rT, b1):
    return pl.pallas_call(
        _merge0_body,
        grid=(GRID,),
        in_specs=[
            pl.BlockSpec((2, RB, H), lambda i: (0, i, 0)),
            pl.BlockSpec((8, H), lambda i: (i, 0)),
            pl.BlockSpec((RB, H), lambda i: (i, 0)),
            pl.BlockSpec((RB, H), lambda i: (i, 0)),
            pl.BlockSpec((H, H), lambda i: (0, 0)),
            pl.BlockSpec((1, H), lambda i: (0, 0)),
            pl.BlockSpec((H, H), lambda i: (0, 0)),
            pl.BlockSpec((H, OUT), lambda i: (0, 0)),
            pl.BlockSpec((1, OUT), lambda i: (0, 0)),
        ],
        out_specs=[
            pl.BlockSpec((RB, H), lambda i: (i, 0)),
            pl.BlockSpec((RB, OUT), lambda i: (i, 0)),
        ],
        out_shape=[
            jax.ShapeDtypeStruct((N, H), jnp.float32),
            jax.ShapeDtypeStruct((N, OUT), jnp.float32),
        ],
    )(feat, cnt, h_paper, h_author, w0lT, b0, w0rT, w1rT, b1)


def _final_body(f_ref, c_ref, r1_ref, w1l_ref, o_ref):
    mean = (f_ref[0] + f_ref[1]) / _cnt_col(c_ref)
    o_ref[...] = (jnp.dot(mean, w1l_ref[...],
                          preferred_element_type=jnp.float32) + r1_ref[...])


def _final(feat, cnt, r1, w1lT):
    return pl.pallas_call(
        _final_body,
        grid=(GRID,),
        in_specs=[
            pl.BlockSpec((2, RB, H), lambda i: (0, i, 0)),
            pl.BlockSpec((8, H), lambda i: (i, 0)),
            pl.BlockSpec((RB, OUT), lambda i: (i, 0)),
            pl.BlockSpec((H, OUT), lambda i: (0, 0)),
        ],
        out_specs=pl.BlockSpec((RB, OUT), lambda i: (i, 0)),
        out_shape=jax.ShapeDtypeStruct((N, OUT), jnp.float32),
    )(feat, cnt, r1, w1lT)


def kernel(x_author, x_paper, edge_index_hop0, edge_index_hop1,
           W_proj_author, b_proj_author, W_proj_paper, b_proj_paper,
           W0_l, b0_l, W0_r, W1_l, b1_l, W1_r):
    E = edge_index_hop0.shape[1]
    EP = NW * NG * GRP * CH
    npad = EP - E
    pad_src = (jnp.arange(npad, dtype=jnp.int32) * 37) % N
    pad_dst = N + (jnp.arange(npad, dtype=jnp.int32) % (NACC - N))

    def prep(idx, pad_vals):
        flat = jnp.concatenate([idx.astype(jnp.int32), pad_vals])
        return flat.reshape(NW * NG, GRP, CH), flat.reshape(EP, 1)

    src0, _ = prep(edge_index_hop0[0], pad_src)
    dst0, dcol0 = prep(edge_index_hop0[1], pad_dst)
    src1, _ = prep(edge_index_hop1[0], pad_src)
    dst1, dcol1 = prep(edge_index_hop1[1], pad_dst)

    zrow = jnp.zeros((CH, H), jnp.float32)

    h_author = _proj(x_author, W_proj_author.T, b_proj_author.reshape(1, H))
    h_paper = _proj(x_paper, W_proj_paper.T, b_proj_paper.reshape(1, H))

    cnt0 = _cnt_tc(dcol0)
    cnt1 = _cnt_tc(dcol1)

    feat0 = _hop_agg(h_author, src0, dst0, zrow)

    h, r1 = _merge0(feat0, cnt0, h_paper, h_author,
                    W0_l.T, b0_l.reshape(1, H), W0_r.T,
                    W1_r.T, b1_l.reshape(1, OUT))

    feat1 = _hop_agg(h, src1, dst1, zrow)

    return _final(feat1, cnt1, r1, W1_l.T)


# speedup vs baseline: 1.0869x; 1.0009x over previous
import functools

import jax
import jax.numpy as jnp
from jax import lax
from jax.experimental import pallas as pl
from jax.experimental.pallas import tpu as pltpu
from jax.experimental.pallas import tpu_sc as plsc

N = 10000
D = 128
H = 128
OUT = 64
CH = 128
NW = 32
NACC = 10240
RPT = NACC // 16
NB = RPT // CH
GRP = 8
NG = 10
RB = 1024
GRID = 10

_MESH = plsc.VectorSubcoreMesh(core_axis_name="c", subcore_axis_name="s")


def _hop_agg(table, srcs, dsts, zrow):
    @functools.partial(
        pl.kernel,
        out_type=jax.ShapeDtypeStruct((2, NACC, H), jnp.float32),
        mesh=_MESH,
        scratch_types=[
            pltpu.VMEM((CH, H), jnp.float32),
            pltpu.VMEM((GRP, CH), jnp.int32),
            pltpu.VMEM((GRP, CH), jnp.int32),
            pltpu.VMEM_SHARED((NACC, H), jnp.float32),
        ],
    )
    def k(table_h, srcs_h, dsts_h, zrow_h, ofeat_h,
          rows_v, src_v, dst_v, acc_s):
        c = lax.axis_index("c")
        s = lax.axis_index("s")
        w = c * 16 + s
        r0 = s * RPT
        pltpu.sync_copy(zrow_h, rows_v)
        for t in range(NB):
            pltpu.sync_copy(rows_v, acc_s.at[pl.ds(r0 + t * CH, CH)])
        plsc.subcore_barrier()

        @pl.loop(0, NG)
        def _(g):
            pltpu.sync_copy(srcs_h.at[w * NG + g], src_v)
            pltpu.sync_copy(dsts_h.at[w * NG + g], dst_v)

            @pl.loop(0, GRP)
            def _(j):
                pltpu.sync_copy(table_h.at[src_v.at[j]], rows_v)
                pltpu.sync_copy(rows_v, acc_s.at[dst_v.at[j]], add=True)

        plsc.subcore_barrier()
        pltpu.sync_copy(acc_s.at[pl.ds(r0, RPT)], ofeat_h.at[c, pl.ds(r0, RPT)])

    return k(table, srcs, dsts, zrow)


def _cnt_body(d_ref, o_ref):
    d = d_ref[...]
    q = jax.lax.shift_right_logical(d, 7)
    r = jax.lax.bitwise_and(d, 127)
    lanes = jax.lax.broadcasted_iota(jnp.int32, (1, H), 1)
    a = jnp.where(q == lanes, 1.0, 0.0)
    b = jnp.where(r == lanes, 1.0, 0.0)
    part = jax.lax.dot_general(a, b, (((0,), (0,)), ((), ())),
                               preferred_element_type=jnp.float32)

    @pl.when(pl.program_id(0) == 0)
    def _():
        o_ref[...] = jnp.zeros_like(o_ref)

    o_ref[...] += part


EC = 8192


def _cnt_tc(dst_col):
    return pl.pallas_call(
        _cnt_body,
        grid=(dst_col.shape[0] // EC,),
        in_specs=[pl.BlockSpec((EC, 1), lambda i: (i, 0))],
        out_specs=pl.BlockSpec((H, H), lambda i: (0, 0)),
        out_shape=jax.ShapeDtypeStruct((H, H), jnp.float32),
    )(dst_col)


def _cnt_col(c_ref):
    m = c_ref[...]
    i0 = jax.lax.broadcasted_iota(jnp.int32, (RB, 8), 0) // H
    s0 = jax.lax.broadcasted_iota(jnp.int32, (RB, 8), 1)
    p = jnp.where(i0 == s0, 1.0, 0.0)
    y = jnp.dot(p, m, preferred_element_type=jnp.float32)
    i1 = jax.lax.broadcasted_iota(jnp.int32, (RB, H), 0) % H
    t1 = jax.lax.broadcasted_iota(jnp.int32, (RB, H), 1)
    qm = jnp.where(i1 == t1, 1.0, 0.0)
    return jnp.maximum(jnp.sum(y * qm, axis=1, keepdims=True), 1.0)


def _proj_body(x_ref, w_ref, b_ref, o_ref):
    o_ref[...] = jnp.maximum(
        jnp.dot(x_ref[...], w_ref[...], preferred_element_type=jnp.float32)
        + b_ref[...], 0.0)


def _proj(x, wT, b):
    return pl.pallas_call(
        _proj_body,
        grid=(GRID,),
        in_specs=[
            pl.BlockSpec((RB, D), lambda i: (i, 0)),
            pl.BlockSpec((D, H), lambda i: (0, 0)),
            pl.BlockSpec((1, H), lambda i: (0, 0)),
        ],
        out_specs=pl.BlockSpec((RB, H), lambda i: (i, 0)),
        out_shape=jax.ShapeDtypeStruct((N, H), jnp.float32),
    )(x, wT, b)


def _merge0_body(f_ref, c_ref, hp_ref, w0l_ref, b0_ref, w0r_ref, h_ref):
    mean = (f_ref[0] + f_ref[1]) / _cnt_col(c_ref)
    t = (jnp.dot(mean, w0l_ref[...], preferred_element_type=jnp.float32)
         + b0_ref[...]
         + jnp.dot(hp_ref[...], w0r_ref[...], preferred_element_type=jnp.float32))
    h_ref[...] = jnp.maximum(t, 0.0)


def _r1_body(ha_ref, w1r_ref, b1_ref, r1_ref):
    r1_ref[...] = (jnp.dot(ha_ref[...], w1r_ref[...],
                           preferred_element_type=jnp.float32) + b1_ref[...])


def _r1(h_author, w1rT, b1):
    return pl.pallas_call(
        _r1_body,
        grid=(GRID,),
        in_specs=[
            pl.BlockSpec((RB, H), lambda i: (i, 0)),
            pl.BlockSpec((H, OUT), lambda i: (0, 0)),
            pl.BlockSpec((1, OUT), lambda i: (0, 0)),
        ],
        out_specs=pl.BlockSpec((RB, OUT), lambda i: (i, 0)),
        out_shape=jax.ShapeDtypeStruct((N, OUT), jnp.float32),
    )(h_author, w1rT, b1)


def _merge0(feat, cnt, h_paper, w0lT, b0, w0rT):
    return pl.pallas_call(
        _merge0_body,
        grid=(GRID,),
        in_specs=[
            pl.BlockSpec((2, RB, H), lambda i: (0, i, 0)),
            pl.BlockSpec((8, H), lambda i: (i, 0)),
            pl.BlockSpec((RB, H), lambda i: (i, 0)),
            pl.BlockSpec((H, H), lambda i: (0, 0)),
            pl.BlockSpec((1, H), lambda i: (0, 0)),
            pl.BlockSpec((H, H), lambda i: (0, 0)),
        ],
        out_specs=pl.BlockSpec((RB, H), lambda i: (i, 0)),
        out_shape=jax.ShapeDtypeStruct((N, H), jnp.float32),
    )(feat, cnt, h_paper, w0lT, b0, w0rT)


def _final_body(f_ref, c_ref, r1_ref, w1l_ref, o_ref):
    mean = (f_ref[0] + f_ref[1]) / _cnt_col(c_ref)
    o_ref[...] = (jnp.dot(mean, w1l_ref[...],
                          preferred_element_type=jnp.float32) + r1_ref[...])


def _final(feat, cnt, r1, w1lT):
    return pl.pallas_call(
        _final_body,
        grid=(GRID,),
        in_specs=[
            pl.BlockSpec((2, RB, H), lambda i: (0, i, 0)),
            pl.BlockSpec((8, H), lambda i: (i, 0)),
            pl.BlockSpec((RB, OUT), lambda i: (i, 0)),
            pl.BlockSpec((H, OUT), lambda i: (0, 0)),
        ],
        out_specs=pl.BlockSpec((RB, OUT), lambda i: (i, 0)),
        out_shape=jax.ShapeDtypeStruct((N, OUT), jnp.float32),
    )(feat, cnt, r1, w1lT)


def kernel(x_author, x_paper, edge_index_hop0, edge_index_hop1,
           W_proj_author, b_proj_author, W_proj_paper, b_proj_paper,
           W0_l, b0_l, W0_r, W1_l, b1_l, W1_r):
    E = edge_index_hop0.shape[1]
    EP = NW * NG * GRP * CH
    npad = EP - E
    pad_src = (jnp.arange(npad, dtype=jnp.int32) * 37) % N
    pad_dst = N + (jnp.arange(npad, dtype=jnp.int32) % (NACC - N))

    def prep(idx, pad_vals):
        flat = jnp.concatenate([idx.astype(jnp.int32), pad_vals])
        return flat.reshape(NW * NG, GRP, CH), flat.reshape(EP, 1)

    src0, _ = prep(edge_index_hop0[0], pad_src)
    dst0, dcol0 = prep(edge_index_hop0[1], pad_dst)
    src1, _ = prep(edge_index_hop1[0], pad_src)
    dst1, dcol1 = prep(edge_index_hop1[1], pad_dst)

    zrow = jnp.zeros((CH, H), jnp.float32)

    h_author = _proj(x_author, W_proj_author.T, b_proj_author.reshape(1, H))
    h_paper = _proj(x_paper, W_proj_paper.T, b_proj_paper.reshape(1, H))

    cnt0 = _cnt_tc(dcol0)
    cnt1 = _cnt_tc(dcol1)

    feat0 = _hop_agg(h_author, src0, dst0, zrow)

    h = _merge0(feat0, cnt0, h_paper, W0_l.T, b0_l.reshape(1, H), W0_r.T)
    r1 = _r1(h_author, W1_r.T, b1_l.reshape(1, OUT))

    feat1 = _hop_agg(h, src1, dst1, zrow)

    return _final(feat1, cnt1, r1, W1_l.T)
